# baseline (speedup 1.0000x reference)
import functools

import jax
import jax.numpy as jnp
from jax import lax
from jax.experimental import pallas as pl
from jax.experimental.pallas import tpu as pltpu
from jax.experimental.pallas import tpu_sc as plsc

VOCAB = 1000000
EMBED = 64
BATCH = 4096
SEQLEN = 200

SC_CORES = 2
SC_SUBCORES = 16
NW = SC_CORES * SC_SUBCORES
BPW = BATCH // NW
HALF = SEQLEN // 2

NCOLS = SEQLEN * EMBED
NTILE = NCOLS // 128
NGRP = BATCH // 8


def _gather_body(xeo_hbm, glove_hbm, out_hbm, idx_v, rows_v, sem):
    wid = lax.axis_index("s") * SC_CORES + lax.axis_index("c")
    pltpu.sync_copy(xeo_hbm.at[wid], idx_v)

    def body(b_local, carry):
        b = wid * BPW + b_local
        b8 = b // 8
        r8 = b % 8
        for h in range(2):
            pltpu.async_copy(
                glove_hbm.at[idx_v.at[2 * b_local + h]], rows_v, sem
            ).wait()
            pltpu.sync_copy(rows_v, out_hbm.at[b8, :, r8, pl.ds(64 * h, 64)])
        return carry

    lax.fori_loop(0, BPW, body, 0)


@functools.cache
def _sc_gather_call():
    return functools.partial(
        pl.kernel,
        mesh=plsc.VectorSubcoreMesh(core_axis_name="c", subcore_axis_name="s"),
        out_type=jax.ShapeDtypeStruct((NGRP, NTILE, 8, 128), jnp.bfloat16),
        scratch_types=[
            pltpu.VMEM((2 * BPW, HALF), jnp.int32),
            pltpu.VMEM((HALF, EMBED), jnp.bfloat16),
            pltpu.SemaphoreType.DMA,
        ],
        compiler_params=pltpu.CompilerParams(use_tc_tiling_on_sc=False),
    )(_gather_body)


def _tc_stats_body(emb_ref, qq_ref, bb_ref, mz_ref, m_ref, z_ref):
    i = pl.program_id(0)

    e3 = emb_ref[...].astype(jnp.float32).reshape(NTILE, 8, 128)
    s3 = qq_ref[...] * e3 + bb_ref[...]
    bm = jnp.max(s3, axis=1)
    m_old = jnp.where(i == 0, bm, m_ref[...])
    z_old = jnp.where(i == 0, 0.0, z_ref[...])
    m_new = jnp.maximum(m_old, bm)
    z_new = z_old * jnp.exp(m_old - m_new) + jnp.sum(
        jnp.exp(s3 - m_new[:, None, :]), axis=1
    )
    z_ref[...] = z_new
    m_ref[...] = m_new

    @pl.when(i == NGRP - 1)
    def _():
        mz_ref[0] = m_new
        mz_ref[1] = jnp.log(z_new)


_tc_stats = pl.pallas_call(
    _tc_stats_body,
    grid=(NGRP,),
    in_specs=[
        pl.BlockSpec((8 * NTILE, 128), lambda i: (i, 0)),
        pl.BlockSpec((1, 128), lambda i: (0, 0)),
        pl.BlockSpec((1, 128), lambda i: (0, 0)),
    ],
    out_specs=pl.BlockSpec((2, NTILE, 128), lambda i: (0, 0, 0)),
    out_shape=jax.ShapeDtypeStruct((2, NTILE, 128), jnp.float32),
    scratch_shapes=[
        pltpu.VMEM((NTILE, 128), jnp.float32),
        pltpu.VMEM((NTILE, 128), jnp.float32),
    ],
    compiler_params=pltpu.CompilerParams(
        dimension_semantics=("arbitrary",),
    ),
)


def _tc_out_body(emb_ref, mz_ref, qq_ref, bb_ref, fcwt_ref, fcb_ref, out_ref):
    e3 = emb_ref[...].astype(jnp.float32).reshape(NTILE, 8, 128)
    s3 = qq_ref[...] * e3 + bb_ref[...]
    lse = mz_ref[0] + mz_ref[1]
    w3 = jnp.exp(s3 - lse[:, None, :]) * e3
    he = jnp.sum(w3, axis=0)
    h = he[:, :EMBED] + he[:, EMBED:]
    out_ref[...] = (
        lax.dot_general(
            h, fcwt_ref[...], (((1,), (0,)), ((), ())),
            preferred_element_type=jnp.float32,
        )
        + fcb_ref[...]
    )


_tc_out = pl.pallas_call(
    _tc_out_body,
    grid=(NGRP,),
    in_specs=[
        pl.BlockSpec((8 * NTILE, 128), lambda i: (i, 0)),
        pl.BlockSpec((2, NTILE, 128), lambda i: (0, 0, 0)),
        pl.BlockSpec((1, 128), lambda i: (0, 0)),
        pl.BlockSpec((1, 128), lambda i: (0, 0)),
        pl.BlockSpec((EMBED, 2), lambda i: (0, 0)),
        pl.BlockSpec((1, 2), lambda i: (0, 0)),
    ],
    out_specs=pl.BlockSpec((8, 2), lambda i: (i, 0)),
    out_shape=jax.ShapeDtypeStruct((BATCH, 2), jnp.float32),
    compiler_params=pltpu.CompilerParams(
        dimension_semantics=("arbitrary",),
    ),
)


def kernel(x, glove, q, bias, fc_w, fc_b):
    xeo = (
        x.astype(jnp.int32)
        .reshape(BATCH, HALF, 2)
        .transpose(0, 2, 1)
        .reshape(NW, 2 * BPW, HALF)
    )
    glove_bf = glove.astype(jnp.bfloat16)
    emb4 = _sc_gather_call()(xeo, glove_bf)
    emb2 = emb4.reshape(NGRP * NTILE * 8, 128)

    qq = jnp.concatenate([q, q]).reshape(1, 128)
    bb = jnp.concatenate([bias, bias]).reshape(1, 128)
    fcwt = fc_w.T
    fcb = fc_b.reshape(1, 2)

    mz = _tc_stats(emb2, qq, bb)
    return _tc_out(emb2, mz, qq, bb, fcwt, fcb)


# speedup vs baseline: 1.4349x; 1.4349x over previous
import functools

import jax
import jax.numpy as jnp
from jax import lax
from jax.experimental import pallas as pl
from jax.experimental.pallas import tpu as pltpu
from jax.experimental.pallas import tpu_sc as plsc

VOCAB = 1000000
EMBED = 64
BATCH = 4096
SEQLEN = 200

SC_CORES = 2
SC_SUBCORES = 16
NW = SC_CORES * SC_SUBCORES
BPW = BATCH // NW
HALF = SEQLEN // 2

NCOLS = SEQLEN * EMBED
NTILE = NCOLS // 128
NGRP = BATCH // 8


def _gather_body(xeo_hbm, glove_hbm, out_hbm, idx_v, rows_a, rows_b, sem):
    wid = lax.axis_index("s") * SC_CORES + lax.axis_index("c")
    pltpu.sync_copy(xeo_hbm.at[wid], idx_v)

    pltpu.async_copy(glove_hbm.at[idx_v.at[0]], rows_a, sem)

    def body(b_local, carry):
        b = wid * BPW + b_local
        b8 = b // 8
        r8 = b % 8
        pltpu.async_copy(glove_hbm.at[idx_v.at[2 * b_local + 1]], rows_b, sem)
        pltpu.make_async_copy(glove_hbm.at[idx_v.at[0]], rows_a, sem).wait()
        pltpu.sync_copy(rows_a, out_hbm.at[b8, :, r8, pl.ds(0, 64)])

        @pl.when(b_local < BPW - 1)
        def _():
            pltpu.async_copy(
                glove_hbm.at[idx_v.at[2 * b_local + 2]], rows_a, sem
            )

        pltpu.make_async_copy(glove_hbm.at[idx_v.at[0]], rows_b, sem).wait()
        pltpu.sync_copy(rows_b, out_hbm.at[b8, :, r8, pl.ds(64, 64)])
        return carry

    lax.fori_loop(0, BPW, body, 0)


@functools.cache
def _sc_gather_call():
    return functools.partial(
        pl.kernel,
        mesh=plsc.VectorSubcoreMesh(core_axis_name="c", subcore_axis_name="s"),
        out_type=jax.ShapeDtypeStruct((NGRP, NTILE, 8, 128), jnp.float32),
        scratch_types=[
            pltpu.VMEM((2 * BPW, HALF), jnp.int32),
            pltpu.VMEM((HALF, EMBED), jnp.float32),
            pltpu.VMEM((HALF, EMBED), jnp.float32),
            pltpu.SemaphoreType.DMA,
        ],
        compiler_params=pltpu.CompilerParams(use_tc_tiling_on_sc=False),
    )(_gather_body)


def _tc_stats_body(emb_ref, qq_ref, bb_ref, mz_ref, m_ref, z_ref):
    i = pl.program_id(0)

    e3 = emb_ref[...].astype(jnp.float32).reshape(NTILE, 8, 128)
    s3 = qq_ref[...] * e3 + bb_ref[...]
    bm = jnp.max(s3, axis=1)
    m_old = jnp.where(i == 0, bm, m_ref[...])
    z_old = jnp.where(i == 0, 0.0, z_ref[...])
    m_new = jnp.maximum(m_old, bm)
    z_new = z_old * jnp.exp(m_old - m_new) + jnp.sum(
        jnp.exp(s3 - m_new[:, None, :]), axis=1
    )
    z_ref[...] = z_new
    m_ref[...] = m_new

    @pl.when(i == NGRP - 1)
    def _():
        mz_ref[0] = m_new
        mz_ref[1] = jnp.log(z_new)


_tc_stats = pl.pallas_call(
    _tc_stats_body,
    grid=(NGRP,),
    in_specs=[
        pl.BlockSpec((8 * NTILE, 128), lambda i: (i, 0)),
        pl.BlockSpec((1, 128), lambda i: (0, 0)),
        pl.BlockSpec((1, 128), lambda i: (0, 0)),
    ],
    out_specs=pl.BlockSpec((2, NTILE, 128), lambda i: (0, 0, 0)),
    out_shape=jax.ShapeDtypeStruct((2, NTILE, 128), jnp.float32),
    scratch_shapes=[
        pltpu.VMEM((NTILE, 128), jnp.float32),
        pltpu.VMEM((NTILE, 128), jnp.float32),
    ],
    compiler_params=pltpu.CompilerParams(
        dimension_semantics=("arbitrary",),
    ),
)


def _tc_out_body(emb_ref, mz_ref, qq_ref, bb_ref, fcwt_ref, fcb_ref, out_ref):
    e3 = emb_ref[...].astype(jnp.float32).reshape(NTILE, 8, 128)
    s3 = qq_ref[...] * e3 + bb_ref[...]
    lse = mz_ref[0] + mz_ref[1]
    w3 = jnp.exp(s3 - lse[:, None, :]) * e3
    he = jnp.sum(w3, axis=0)
    h = he[:, :EMBED] + he[:, EMBED:]
    out_ref[...] = (
        lax.dot_general(
            h, fcwt_ref[...], (((1,), (0,)), ((), ())),
            preferred_element_type=jnp.float32,
        )
        + fcb_ref[...]
    )


_tc_out = pl.pallas_call(
    _tc_out_body,
    grid=(NGRP,),
    in_specs=[
        pl.BlockSpec((8 * NTILE, 128), lambda i: (i, 0)),
        pl.BlockSpec((2, NTILE, 128), lambda i: (0, 0, 0)),
        pl.BlockSpec((1, 128), lambda i: (0, 0)),
        pl.BlockSpec((1, 128), lambda i: (0, 0)),
        pl.BlockSpec((EMBED, 2), lambda i: (0, 0)),
        pl.BlockSpec((1, 2), lambda i: (0, 0)),
    ],
    out_specs=pl.BlockSpec((8, 2), lambda i: (i, 0)),
    out_shape=jax.ShapeDtypeStruct((BATCH, 2), jnp.float32),
    compiler_params=pltpu.CompilerParams(
        dimension_semantics=("arbitrary",),
    ),
)


def kernel(x, glove, q, bias, fc_w, fc_b):
    xeo = (
        x.astype(jnp.int32)
        .reshape(BATCH, HALF, 2)
        .transpose(0, 2, 1)
        .reshape(NW, 2 * BPW, HALF)
    )
    emb4 = _sc_gather_call()(xeo, glove)
    emb2 = emb4.reshape(NGRP * NTILE * 8, 128)

    qq = jnp.concatenate([q, q]).reshape(1, 128)
    bb = jnp.concatenate([bias, bias]).reshape(1, 128)
    fcwt = fc_w.T
    fcb = fc_b.reshape(1, 2)

    mz = _tc_stats(emb2, qq, bb)
    return _tc_out(emb2, mz, qq, bb, fcwt, fcb)
